# baseline (speedup 1.0000x reference)
import jax
import jax.numpy as jnp
from jax import lax
from jax.experimental import pallas as pl
from jax.experimental.pallas import tpu as pltpu
from jax.experimental.pallas import tpu_sc as plsc

B_TOTAL = 16384 * 50
D = 64
NC, NS = 2, 16
NW = NC * NS
B_PER_W = B_TOTAL // NW
K = 128
G = 8
ROWS_PER_GROUP = K * G
N_GROUPS = B_PER_W // ROWS_PER_GROUP
N_STREAMS = B_PER_W // K


def _emb_body(idx_hbm, table_hbm, out_hbm, idx_v, rows_v, gsem):
    wid = lax.axis_index("s") * NC + lax.axis_index("c")
    base = wid * B_PER_W

    pltpu.sync_copy(idx_hbm.at[pl.ds(wid * N_STREAMS, N_STREAMS), :], idx_v)

    def group(g, _):
        row_base = g * G
        copies = []
        for j in range(G):
            cp = pltpu.async_copy(
                table_hbm.at[idx_v.at[row_base + j]],
                rows_v.at[pl.ds(j * K, K), :],
                gsem,
            )
            copies.append(cp)
        for cp in copies:
            cp.wait()
        out_off = pl.multiple_of(base + g * ROWS_PER_GROUP, 8)
        pltpu.sync_copy(rows_v, out_hbm.at[pl.ds(out_off, ROWS_PER_GROUP), :])
        return 0

    lax.fori_loop(0, N_GROUPS, group, 0)


@jax.jit
def _emb(x_flat2d, weight):
    mesh = plsc.VectorSubcoreMesh(core_axis_name="c", subcore_axis_name="s")
    run = pl.kernel(
        _emb_body,
        mesh=mesh,
        out_type=jax.ShapeDtypeStruct((B_TOTAL, D), jnp.float32),
        scratch_types=[
            pltpu.VMEM((N_STREAMS, K), jnp.int32),
            pltpu.VMEM((ROWS_PER_GROUP, D), jnp.float32),
            pltpu.SemaphoreType.DMA,
        ],
        compiler_params=pltpu.CompilerParams(use_tc_tiling_on_sc=False),
    )
    return run(x_flat2d, weight)


def kernel(x, weight):
    x_flat = x.reshape(-1).astype(jnp.int32).reshape(NW * N_STREAMS, K)
    out = _emb(x_flat, weight)
    return out.reshape(x.shape[0], x.shape[1], D)


# speedup vs baseline: 1.0040x; 1.0040x over previous
import jax
import jax.numpy as jnp
from jax import lax
from jax.experimental import pallas as pl
from jax.experimental.pallas import tpu as pltpu
from jax.experimental.pallas import tpu_sc as plsc

B_TOTAL = 16384 * 50
D = 64
NC, NS = 2, 16
NW = NC * NS
B_PER_W = B_TOTAL // NW
K = 128
G = 5
ROWS_PER_GROUP = K * G
N_GROUPS = B_PER_W // ROWS_PER_GROUP
N_STREAMS = B_PER_W // K


def _emb_body(idx_hbm, table_hbm, out_hbm, idx_v, rows0, rows1,
              gsem0, gsem1, wsem0, wsem1):
    wid = lax.axis_index("s") * NC + lax.axis_index("c")
    base = wid * B_PER_W
    rows = (rows0, rows1)
    gsem = (gsem0, gsem1)
    wsem = (wsem0, wsem1)

    pltpu.sync_copy(idx_hbm.at[pl.ds(wid * N_STREAMS, N_STREAMS), :], idx_v)

    def fire_gathers(g, b):
        row_base = g * G
        for j in range(G):
            pltpu.async_copy(
                table_hbm.at[idx_v.at[row_base + j]],
                rows[b].at[pl.ds(j * K, K), :],
                gsem[b],
            )

    def drain_gathers(b):
        pltpu.make_async_copy(
            table_hbm.at[pl.ds(0, ROWS_PER_GROUP), :], rows[b], gsem[b]
        ).wait()

    def fire_write(g, b):
        out_off = pl.multiple_of(base + g * ROWS_PER_GROUP, 8)
        pltpu.async_copy(
            rows[b], out_hbm.at[pl.ds(out_off, ROWS_PER_GROUP), :], wsem[b]
        )

    def drain_write(b):
        pltpu.make_async_copy(
            rows[b], out_hbm.at[pl.ds(0, ROWS_PER_GROUP), :], wsem[b]
        ).wait()

    fire_gathers(0, 0)
    fire_gathers(1, 1)

    def step(i, _):
        t = 2 * i
        drain_gathers(0)
        fire_write(t, 0)
        drain_gathers(1)
        fire_write(t + 1, 1)
        drain_write(0)
        fire_gathers(t + 2, 0)
        drain_write(1)
        fire_gathers(t + 3, 1)
        return 0

    lax.fori_loop(0, (N_GROUPS - 2) // 2, step, 0)

    drain_gathers(0)
    fire_write(N_GROUPS - 2, 0)
    drain_gathers(1)
    fire_write(N_GROUPS - 1, 1)
    drain_write(0)
    drain_write(1)


@jax.jit
def _emb(x_flat2d, weight):
    mesh = plsc.VectorSubcoreMesh(core_axis_name="c", subcore_axis_name="s")
    run = pl.kernel(
        _emb_body,
        mesh=mesh,
        out_type=jax.ShapeDtypeStruct((B_TOTAL, D), jnp.float32),
        scratch_types=[
            pltpu.VMEM((N_STREAMS, K), jnp.int32),
            pltpu.VMEM((ROWS_PER_GROUP, D), jnp.float32),
            pltpu.VMEM((ROWS_PER_GROUP, D), jnp.float32),
            pltpu.SemaphoreType.DMA,
            pltpu.SemaphoreType.DMA,
            pltpu.SemaphoreType.DMA,
            pltpu.SemaphoreType.DMA,
        ],
        compiler_params=pltpu.CompilerParams(use_tc_tiling_on_sc=False),
    )
    return run(x_flat2d, weight)


def kernel(x, weight):
    x_flat = x.reshape(-1).astype(jnp.int32).reshape(NW * N_STREAMS, K)
    out = _emb(x_flat, weight)
    return out.reshape(x.shape[0], x.shape[1], D)
